# baseline (speedup 1.0000x reference)
import jax
import jax.numpy as jnp
from jax import lax
from jax.experimental import pallas as pl
from jax.experimental.pallas import tpu as pltpu
from jax.experimental.pallas import tpu_sc as plsc

WD = 1e-4
B = 16384
D = 64
L = 16
NC = 2
NS = 16
NW = NC * NS
BPW = B // NW
G = 16
NG = BPW // G
NROW = 1000000
NBLK = NROW // 8
BKC = 8192
NGRID = (NROW + BKC - 1) // BKC


def _norms_body(tu_ref, ti_ref, ou_ref, oi_ref):
    b = pl.program_id(0)
    col = lax.broadcasted_iota(jnp.int32, (D, BKC), 1) + b * BKC
    m = col < NROW
    xu = jnp.where(m, tu_ref[...], 0.0)
    xi = jnp.where(m, ti_ref[...], 0.0)
    ou_ref[...] = jnp.sum(xu * xu, axis=0)
    oi_ref[...] = jnp.sum(xi * xi, axis=0)


def _sc_body(emb_u2, emb_i2, users, pos, neg, nrm_u, nrm_i,
             scores_out, sq_out,
             idxu, idxp, idxn, blku, blkp, blkn,
             bu, bp, bn, nbu, nbp, nbn, scb, sqb, sem, sem2):
    cid = lax.axis_index("c")
    sid = lax.axis_index("s")
    wid = sid * NC + cid
    base = wid * BPW

    pltpu.sync_copy(users.at[pl.ds(base, BPW)], idxu)
    pltpu.sync_copy(pos.at[pl.ds(base, BPW)], idxp)
    pltpu.sync_copy(neg.at[pl.ds(base, BPW)], idxn)

    ncps = []
    for k in range(BPW // 128):
        s128 = pl.ds(k * 128, 128)
        ncps.append(pltpu.async_copy(
            nrm_u.at[idxu.at[s128]], nbu.at[s128], sem2))
        ncps.append(pltpu.async_copy(
            nrm_i.at[idxp.at[s128]], nbp.at[s128], sem2))
        ncps.append(pltpu.async_copy(
            nrm_i.at[idxn.at[s128]], nbn.at[s128], sem2))

    def group(c, carry):
        s16 = pl.ds(c * G, G)
        ivu = idxu[s16]
        ivp = idxp[s16]
        ivn = idxn[s16]
        blku[...] = lax.shift_right_logical(ivu, 1)
        blkp[...] = lax.shift_right_logical(ivp, 1)
        blkn[...] = lax.shift_right_logical(ivn, 1)
        cps = [
            pltpu.async_copy(emb_u2.at[blku], bu, sem),
            pltpu.async_copy(emb_i2.at[blkp], bp, sem),
            pltpu.async_copy(emb_i2.at[blkn], bn, sem),
        ]
        for cp in cps:
            cp.wait()

        for j in range(G):
            ou = jnp.bitwise_and(ivu[j], 1) * D
            op = jnp.bitwise_and(ivp[j], 1) * D
            on = jnp.bitwise_and(ivn[j], 1) * D
            acc = jnp.zeros((L,), jnp.float32)
            for k in range(D // L):
                uv = bu[j, pl.ds(ou + k * L, L)]
                pv = bp[j, pl.ds(op + k * L, L)]
                nv = bn[j, pl.ds(on + k * L, L)]
                acc = acc + uv * (nv - pv)
            scb[j, pl.ds(0, L)] = acc
        pltpu.sync_copy(scb, scores_out.at[pl.ds(base + c * G, G)])
        return carry

    lax.fori_loop(0, NG, group, 0)

    for cp in ncps:
        cp.wait()
    sqv = jnp.zeros((L,), jnp.float32)
    for k in range(BPW // L):
        s = pl.ds(k * L, L)
        sqv = sqv + nbu[s] + nbp[s] + nbn[s]
    sqb[...] = sqv
    pltpu.sync_copy(sqb, sq_out.at[pl.ds(wid * L, L)])


def _tc_body(sc_ref, sq_ref, out_ref):
    x = jnp.sum(sc_ref[:, 0:L], axis=1, keepdims=True)
    sp = jnp.maximum(x, 0.0) + jnp.log1p(jnp.exp(-jnp.abs(x)))
    reg = jnp.sum(sq_ref[...])
    out_ref[0, 0] = jnp.sum(sp) / B + (0.5 * WD / B) * reg


def kernel(emb_users, emb_items, users, pos_items, neg_items,
           raw_emb_users, raw_emb_items):
    users = users.astype(jnp.int32)
    pos_items = pos_items.astype(jnp.int32)
    neg_items = neg_items.astype(jnp.int32)
    emb_u2 = emb_users.reshape(NROW // 2, 2 * D)
    emb_i2 = emb_items.reshape(NROW // 2, 2 * D)

    nrm_u, nrm_i = pl.pallas_call(
        _norms_body,
        grid=(NGRID,),
        in_specs=[
            pl.BlockSpec((D, BKC), lambda b: (0, b)),
            pl.BlockSpec((D, BKC), lambda b: (0, b)),
        ],
        out_specs=[
            pl.BlockSpec((BKC,), lambda b: (b,)),
            pl.BlockSpec((BKC,), lambda b: (b,)),
        ],
        out_shape=[
            jax.ShapeDtypeStruct((NROW,), jnp.float32),
            jax.ShapeDtypeStruct((NROW,), jnp.float32),
        ],
    )(raw_emb_users.T, raw_emb_items.T)

    mesh = plsc.VectorSubcoreMesh(
        core_axis_name="c", subcore_axis_name="s",
        num_cores=NC, num_subcores=NS)
    sc = pl.kernel(
        _sc_body,
        out_type=[
            jax.ShapeDtypeStruct((B, 128), jnp.float32),
            jax.ShapeDtypeStruct((NW * L,), jnp.float32),
        ],
        mesh=mesh,
        scratch_types=[
            pltpu.VMEM((BPW,), jnp.int32),
            pltpu.VMEM((BPW,), jnp.int32),
            pltpu.VMEM((BPW,), jnp.int32),
            pltpu.VMEM((G,), jnp.int32),
            pltpu.VMEM((G,), jnp.int32),
            pltpu.VMEM((G,), jnp.int32),
            pltpu.VMEM((G, 128), jnp.float32),
            pltpu.VMEM((G, 128), jnp.float32),
            pltpu.VMEM((G, 128), jnp.float32),
            pltpu.VMEM((BPW,), jnp.float32),
            pltpu.VMEM((BPW,), jnp.float32),
            pltpu.VMEM((BPW,), jnp.float32),
            pltpu.VMEM((G, 128), jnp.float32),
            pltpu.VMEM((L,), jnp.float32),
            pltpu.SemaphoreType.DMA,
            pltpu.SemaphoreType.DMA,
        ],
    )
    scores, sq = sc(emb_u2, emb_i2, users, pos_items, neg_items,
                    nrm_u, nrm_i)

    out = pl.pallas_call(
        _tc_body,
        out_shape=jax.ShapeDtypeStruct((1, 1), jnp.float32),
        out_specs=pl.BlockSpec(memory_space=pltpu.SMEM),
    )(scores, sq.reshape(4, 128))
    return out[0, 0]


# speedup vs baseline: 1.9140x; 1.9140x over previous
import jax
import jax.numpy as jnp
from jax import lax
from jax.experimental import pallas as pl
from jax.experimental.pallas import tpu as pltpu
from jax.experimental.pallas import tpu_sc as plsc

WD = 1e-4
B = 16384
D = 64
L = 16
NC = 2
NS = 16
NW = NC * NS
BPW = B // NW
G = 16
NG = BPW // G
NROW = 1000000
NBLK = NROW // 8
BKC = 16384
NGRID = (NROW + BKC - 1) // BKC


def _norms_body(tu_ref, ti_ref, ou_ref, oi_ref):
    xu = tu_ref[...]
    xi = ti_ref[...]
    ou_ref[...] = jnp.sum(xu * xu, axis=0)
    oi_ref[...] = jnp.sum(xi * xi, axis=0)


def _sc_body(emb_u3, emb_i3, users, pos, neg, nrm_u, nrm_i,
             scores_out, sq_out,
             idxu, idxp, idxn,
             bu, bp, bn, nbu, nbp, nbn, scb, sqb, sem, sem2):
    cid = lax.axis_index("c")
    sid = lax.axis_index("s")
    wid = sid * NC + cid
    base = wid * BPW

    pltpu.sync_copy(users.at[pl.ds(base, BPW)], idxu)
    pltpu.sync_copy(pos.at[pl.ds(base, BPW)], idxp)
    pltpu.sync_copy(neg.at[pl.ds(base, BPW)], idxn)

    ncps = []
    for k in range(BPW // 128):
        s128 = pl.ds(k * 128, 128)
        ncps.append(pltpu.async_copy(
            nrm_u.at[idxu.at[s128]], nbu.at[s128], sem2))
        ncps.append(pltpu.async_copy(
            nrm_i.at[idxp.at[s128]], nbp.at[s128], sem2))
        ncps.append(pltpu.async_copy(
            nrm_i.at[idxn.at[s128]], nbn.at[s128], sem2))

    def group(c, carry):
        s16 = pl.ds(c * G, G)
        ivu = idxu[s16]
        ivp = idxp[s16]
        ivn = idxn[s16]
        bkvu = lax.shift_right_logical(ivu, 3)
        bkvp = lax.shift_right_logical(ivp, 3)
        bkvn = lax.shift_right_logical(ivn, 3)
        cps = []
        for j in range(G):
            cps.append(pltpu.async_copy(emb_u3.at[bkvu[j]], bu.at[j], sem))
            cps.append(pltpu.async_copy(emb_i3.at[bkvp[j]], bp.at[j], sem))
            cps.append(pltpu.async_copy(emb_i3.at[bkvn[j]], bn.at[j], sem))
        for cp in cps:
            cp.wait()

        for j in range(G):
            ru = jnp.bitwise_and(ivu[j], 7)
            rp = jnp.bitwise_and(ivp[j], 7)
            rn = jnp.bitwise_and(ivn[j], 7)
            acc = jnp.zeros((L,), jnp.float32)
            for k in range(D // L):
                s = pl.ds(k * L, L)
                uv = bu[j, ru, s]
                pv = bp[j, rp, s]
                nv = bn[j, rn, s]
                acc = acc + uv * (nv - pv)
            scb[j, pl.ds(0, L)] = acc
        pltpu.sync_copy(scb, scores_out.at[pl.ds(base + c * G, G)])
        return carry

    lax.fori_loop(0, NG, group, 0)

    for cp in ncps:
        cp.wait()
    sqv = jnp.zeros((L,), jnp.float32)
    for k in range(BPW // L):
        s = pl.ds(k * L, L)
        sqv = sqv + nbu[s] + nbp[s] + nbn[s]
    sqb[...] = sqv
    pltpu.sync_copy(sqb, sq_out.at[pl.ds(wid * L, L)])


def _tc_body(sc_ref, sq_ref, out_ref):
    x = jnp.sum(sc_ref[:, 0:L], axis=1, keepdims=True)
    sp = jnp.maximum(x, 0.0) + jnp.log1p(jnp.exp(-jnp.abs(x)))
    reg = jnp.sum(sq_ref[...])
    out_ref[0, 0] = jnp.sum(sp) / B + (0.5 * WD / B) * reg


def kernel(emb_users, emb_items, users, pos_items, neg_items,
           raw_emb_users, raw_emb_items):
    users = users.astype(jnp.int32)
    pos_items = pos_items.astype(jnp.int32)
    neg_items = neg_items.astype(jnp.int32)
    emb_u3 = emb_users.reshape(NBLK, 8, D)
    emb_i3 = emb_items.reshape(NBLK, 8, D)

    nrm_u, nrm_i = pl.pallas_call(
        _norms_body,
        grid=(NGRID,),
        in_specs=[
            pl.BlockSpec((D, BKC), lambda b: (0, b)),
            pl.BlockSpec((D, BKC), lambda b: (0, b)),
        ],
        out_specs=[
            pl.BlockSpec((BKC,), lambda b: (b,)),
            pl.BlockSpec((BKC,), lambda b: (b,)),
        ],
        out_shape=[
            jax.ShapeDtypeStruct((NROW,), jnp.float32),
            jax.ShapeDtypeStruct((NROW,), jnp.float32),
        ],
    )(raw_emb_users.T, raw_emb_items.T)

    mesh = plsc.VectorSubcoreMesh(
        core_axis_name="c", subcore_axis_name="s",
        num_cores=NC, num_subcores=NS)
    sc = pl.kernel(
        _sc_body,
        out_type=[
            jax.ShapeDtypeStruct((B, 128), jnp.float32),
            jax.ShapeDtypeStruct((NW * L,), jnp.float32),
        ],
        mesh=mesh,
        scratch_types=[
            pltpu.VMEM((BPW,), jnp.int32),
            pltpu.VMEM((BPW,), jnp.int32),
            pltpu.VMEM((BPW,), jnp.int32),
            pltpu.VMEM((G, 8, D), jnp.float32),
            pltpu.VMEM((G, 8, D), jnp.float32),
            pltpu.VMEM((G, 8, D), jnp.float32),
            pltpu.VMEM((BPW,), jnp.float32),
            pltpu.VMEM((BPW,), jnp.float32),
            pltpu.VMEM((BPW,), jnp.float32),
            pltpu.VMEM((G, 128), jnp.float32),
            pltpu.VMEM((L,), jnp.float32),
            pltpu.SemaphoreType.DMA,
            pltpu.SemaphoreType.DMA,
        ],
    )
    scores, sq = sc(emb_u3, emb_i3, users, pos_items, neg_items,
                    nrm_u, nrm_i)

    out = pl.pallas_call(
        _tc_body,
        out_shape=jax.ShapeDtypeStruct((1, 1), jnp.float32),
        out_specs=pl.BlockSpec(memory_space=pltpu.SMEM),
    )(scores, sq.reshape(4, 128))
    return out[0, 0]
